# baseline (speedup 1.0000x reference)
import functools

import jax
import jax.numpy as jnp
from jax import lax
from jax.experimental import pallas as pl
from jax.experimental.pallas import tpu as pltpu
from jax.experimental.pallas import tpu_sc as plsc

B = 16
H = W = 384
NPTS = 512
L = 16
CHUNK = 128
NCHUNKS = NPTS // CHUNK
VECS = CHUNK // L
IMG = H * W
SCALE = 512.0

_mesh = plsc.VectorSubcoreMesh(
    core_axis_name="c", subcore_axis_name="s", num_cores=1
)


@functools.partial(
    pl.kernel,
    mesh=_mesh,
    out_type=jax.ShapeDtypeStruct((L,), jnp.float32),
    scratch_types=[
        pltpu.VMEM((NPTS,), jnp.float32),
        pltpu.VMEM((NPTS,), jnp.float32),
        pltpu.VMEM((NPTS,), jnp.float32),
        pltpu.VMEM((CHUNK,), jnp.int32),
        pltpu.VMEM((CHUNK,), jnp.float32),
        pltpu.VMEM((L,), jnp.float32),
        pltpu.SMEM((2,), jnp.int32),
        pltpu.SemaphoreType.DMA,
    ],
    compiler_params=pltpu.CompilerParams(needs_layout_passes=False),
)
def _depth_loss_kernel(
    img_hbm, rows_hbm, cols_hbm, dep_hbm, out_hbm,
    rows_v, cols_v, dep_v, idx_v, val_v, res_v, smem, sem,
):
    wid = lax.axis_index("s")

    @pl.when(wid == 0)
    def _():
        smem[0] = 0
        smem[1] = 0

    plsc.subcore_barrier()

    pltpu.sync_copy(rows_hbm.at[pl.ds(wid * NPTS, NPTS)], rows_v)
    pltpu.sync_copy(cols_hbm.at[pl.ds(wid * NPTS, NPTS)], cols_v)
    pltpu.sync_copy(dep_hbm.at[pl.ds(wid * NPTS, NPTS)], dep_v)

    base = wid * IMG
    acc = jnp.zeros((L,), jnp.float32)
    cnt = jnp.zeros((L,), jnp.float32)
    for ch in range(NCHUNKS):
        for v in range(VECS):
            off = ch * CHUNK + v * L
            ri = rows_v[pl.ds(off, L)].astype(jnp.int32)
            ci = cols_v[pl.ds(off, L)].astype(jnp.int32)
            idx_v[pl.ds(v * L, L)] = base + ri * W + ci
        pltpu.async_copy(img_hbm.at[idx_v], val_v, sem).wait()
        for v in range(VECS):
            off = ch * CHUNK + v * L
            d = dep_v[pl.ds(off, L)]
            g = val_v[pl.ds(v * L, L)]
            m = d > 0.0
            acc = acc + jnp.where(m, jnp.abs(g - d), 0.0)
            cnt = cnt + jnp.where(m, 1.0, 0.0)

    s_i = jnp.sum((acc * SCALE + 0.5).astype(jnp.int32))
    c_i = jnp.sum(cnt.astype(jnp.int32))
    plsc.fetch_and_add(smem.at[0], s_i, subcore_id=0)
    plsc.fetch_and_add(smem.at[1], c_i, subcore_id=0)
    plsc.subcore_barrier()

    @pl.when(wid == 0)
    def _():
        sv = jnp.full((L,), smem[0], jnp.int32).astype(jnp.float32) * (1.0 / SCALE)
        cv = jnp.full((L,), smem[1], jnp.int32).astype(jnp.float32)
        lossv = jnp.where(
            cv > 0.0, sv / jnp.maximum(cv, 1.0), jnp.zeros((L,), jnp.float32)
        )
        res_v[...] = lossv
        pltpu.sync_copy(res_v, out_hbm)


@jax.jit
def kernel(output, rdepth):
    img = output.reshape(-1)
    rows = rdepth[:, :, 0].reshape(-1)
    cols = rdepth[:, :, 1].reshape(-1)
    dep = rdepth[:, :, 2].reshape(-1)
    res = _depth_loss_kernel(img, rows, cols, dep)
    return res[0]
